# baseline (speedup 1.0000x reference)
import jax
import jax.numpy as jnp
from jax.experimental import pallas as pl


def _add_row_kernel(x_ref, p_ref, o_ref):
    o_ref[...] = x_ref[...] + p_ref[...]


def kernel(x, pos_table):
    B, T, E = x.shape
    TB = 512
    grid = (B, T // TB)
    pt3 = pos_table.reshape(pos_table.shape[0], 1, E)
    return pl.pallas_call(
        _add_row_kernel,
        grid=grid,
        in_specs=[
            pl.BlockSpec((1, TB, E), lambda b, t: (b, t, 0)),
            pl.BlockSpec((1, 1, E), lambda b, t: (b, 0, 0)),
        ],
        out_specs=pl.BlockSpec((1, TB, E), lambda b, t: (b, t, 0)),
        out_shape=jax.ShapeDtypeStruct((B, T, E), x.dtype),
    )(x, pt3)


# speedup vs baseline: 1.0848x; 1.0848x over previous
import jax
import jax.numpy as jnp
from jax.experimental import pallas as pl


def _add_row_kernel(x_ref, p_ref, o_ref):
    o_ref[...] = x_ref[...] + p_ref[...]


def kernel(x, pos_table):
    B, T, E = x.shape
    TB = 2048
    grid = (B, T // TB)
    pt3 = pos_table.reshape(pos_table.shape[0], 1, E)
    return pl.pallas_call(
        _add_row_kernel,
        grid=grid,
        in_specs=[
            pl.BlockSpec((1, TB, E), lambda b, t: (b, t, 0)),
            pl.BlockSpec((1, 1, E), lambda b, t: (b, 0, 0)),
        ],
        out_specs=pl.BlockSpec((1, TB, E), lambda b, t: (b, t, 0)),
        out_shape=jax.ShapeDtypeStruct((B, T, E), x.dtype),
    )(x, pt3)
